# baseline (speedup 1.0000x reference)
import functools

import jax
import jax.numpy as jnp
from jax import lax
from jax.experimental import pallas as pl
from jax.experimental.pallas import tpu as pltpu
from jax.experimental.pallas import tpu_sc as plsc

_BATCH = 16384
_D = 32
_R = 5
_NC = 2
_NS = 16
_NW = _NC * _NS
_EPW = _BATCH // _NW
_C = 128
_NCHUNK = _EPW // _C
_RC = _R * _C
_F = 6 * _D
_NV = _F // 16

_PN = 1000000
_TN = 100000
_TBLK = 512
_PNB = _PN // _TBLK
_PREM = _PN - _PNB * _TBLK
_TNB = _TN // _TBLK
_TREM = _TN - _TNB * _TBLK


def _ceil_div(a, b):
  return -(-a // b)


def _build_transpose_kernel():
  mesh = plsc.VectorSubcoreMesh(core_axis_name="c", subcore_axis_name="s")
  pni2 = _ceil_div(_ceil_div(_PNB, _NW), 2)
  tni2 = _ceil_div(_ceil_div(_TNB, _NW), 2)

  @functools.partial(
      pl.kernel,
      out_type=(jax.ShapeDtypeStruct((_PN * _D,), jnp.float32),
                jax.ShapeDtypeStruct((_TN * _D,), jnp.float32)),
      mesh=mesh,
      compiler_params=pltpu.CompilerParams(
          use_tc_tiling_on_sc=True, needs_layout_passes=False),
      scratch_types=[
          pltpu.VMEM((_D, _TBLK), jnp.float32),
          pltpu.VMEM((_D, _TBLK), jnp.float32),
          pltpu.VMEM((_D * _TBLK,), jnp.float32),
          pltpu.VMEM((_D * _TBLK,), jnp.float32),
          pltpu.SemaphoreType.DMA,
          pltpu.SemaphoreType.DMA,
          pltpu.SemaphoreType.DMA,
          pltpu.SemaphoreType.DMA,
      ],
  )
  def k(pt_h, tt_h, ptail_h, ttail_h, pout_h, tout_h,
        inb0, inb1, outb0, outb1, si0, si1, so0, so1):
    wid = lax.axis_index("s") * _NC + lax.axis_index("c")
    inb = (inb0, inb1)
    outb = (outb0, outb1)
    si = (si0, si1)
    so = (so0, so1)
    lane32 = lax.iota(jnp.int32, 16) * _D

    def transpose_block(bi, bo):
      def fbody(f, c):
        base = lane32 + f
        for g in range(_TBLK // 16):
          v = bi[f, pl.ds(16 * g, 16)]
          plsc.store_scatter(bo, [base + g * 16 * _D], v)
        return c
      lax.fori_loop(0, _D, fbody, 0)

    def run_table(src_h, dst_h, nb, ni2):
      def blk(step):
        return jnp.minimum(wid + step * _NW, nb - 1)

      def issue_in(step, b):
        off = pl.multiple_of(blk(step) * _TBLK, _TBLK)
        pltpu.async_copy(src_h.at[:, pl.ds(off, _TBLK)], inb[b], si[b])

      issue_in(0, 0)
      issue_in(1, 1)

      def outer(i, c):
        for b in (0, 1):
          step = 2 * i + b
          pltpu.make_async_copy(
              src_h.at[:, pl.ds(0, _TBLK)], inb[b], si[b]).wait()

          @pl.when(i > 0)
          def _():
            pltpu.make_async_copy(
                outb[b], dst_h.at[pl.ds(0, _TBLK * _D)], so[b]).wait()

          transpose_block(inb[b], outb[b])
          off = pl.multiple_of(blk(step) * _TBLK * _D, _TBLK * _D)
          pltpu.async_copy(outb[b], dst_h.at[pl.ds(off, _TBLK * _D)], so[b])

          @pl.when(i < ni2 - 1)
          def _():
            issue_in(step + 2, b)
        return c

      lax.fori_loop(0, ni2, outer, 0)
      for b in (0, 1):
        pltpu.make_async_copy(
            outb[b], dst_h.at[pl.ds(0, _TBLK * _D)], so[b]).wait()

    run_table(pt_h, pout_h, _PNB, pni2)
    run_table(tt_h, tout_h, _TNB, tni2)

    @pl.when(wid == 5)
    def _():
      n = _PREM * _D
      pltpu.sync_copy(ptail_h, outb0.at[pl.ds(0, n)])
      pltpu.sync_copy(outb0.at[pl.ds(0, n)],
                      pout_h.at[pl.ds(_PNB * _TBLK * _D, n)])

    @pl.when(wid == 6)
    def _():
      n = _TREM * _D
      pltpu.sync_copy(ttail_h, outb1.at[pl.ds(0, n)])
      pltpu.sync_copy(outb1.at[pl.ds(0, n)],
                      tout_h.at[pl.ds(_TNB * _TBLK * _D, n)])

  return k


def _lanesum(v):
  dnums = lax.GatherDimensionNumbers(
      offset_dims=(), collapsed_slice_dims=(0,), start_index_map=(0,))
  lane = lax.iota(jnp.int32, 16)
  for sh in (8, 4, 2, 1):
    perm = lax.bitwise_xor(lane, sh)
    shuf = lax.gather(v, perm[:, None], dnums, slice_sizes=(1,),
                      mode=lax.GatherScatterMode.PROMISE_IN_BOUNDS)
    v = v + shuf
  return v


def _build_gather_kernel():
  mesh = plsc.VectorSubcoreMesh(core_axis_name="c", subcore_axis_name="s")

  @functools.partial(
      pl.kernel,
      out_type=jax.ShapeDtypeStruct((2, _BATCH), jnp.float32),
      mesh=mesh,
      compiler_params=pltpu.CompilerParams(use_tc_tiling_on_sc=False),
      scratch_types=[
          pltpu.VMEM((_C,), jnp.int32),
          pltpu.VMEM((_C,), jnp.int32),
          pltpu.VMEM((_RC,), jnp.int32),
          pltpu.VMEM((_RC,), jnp.int32),
          pltpu.VMEM((_RC,), jnp.int32),
          pltpu.VMEM((_RC,), jnp.int32),
          pltpu.VMEM((_C, _D), jnp.float32),
          pltpu.VMEM((_C, _D), jnp.float32),
          pltpu.VMEM((_RC, _D), jnp.float32),
          pltpu.VMEM((_RC, _D), jnp.float32),
          pltpu.VMEM((_RC, _D), jnp.float32),
          pltpu.VMEM((_RC, _D), jnp.float32),
          pltpu.VMEM((2, _F), jnp.float32),
          pltpu.VMEM((2, _C), jnp.float32),
          pltpu.SemaphoreType.DMA,
      ],
  )
  def k(t1i_h, t2i_h, c1i_h, c2i_h, p1i_h, p2i_h,
        temb_h, cemb_h, pemb_h, w_h, out_h,
        t1i, t2i, c1i, c2i, p1i, p2i,
        t1r, t2r, c1r, c2r, p1r, p2r, wv, ob, sem):
    wid = lax.axis_index("s") * _NC + lax.axis_index("c")
    pltpu.sync_copy(w_h, wv)

    def chunk_body(g, carry):
      base = wid * _EPW + g * _C
      rbase = base * _R

      pltpu.sync_copy(t1i_h.at[pl.ds(base, _C)], t1i)
      pltpu.sync_copy(t2i_h.at[pl.ds(base, _C)], t2i)
      pltpu.sync_copy(c1i_h.at[pl.ds(rbase, _RC)], c1i)
      pltpu.sync_copy(c2i_h.at[pl.ds(rbase, _RC)], c2i)
      pltpu.sync_copy(p1i_h.at[pl.ds(rbase, _RC)], p1i)
      pltpu.sync_copy(p2i_h.at[pl.ds(rbase, _RC)], p2i)

      copies = [
          pltpu.async_copy(temb_h.at[t1i], t1r, sem),
          pltpu.async_copy(temb_h.at[t2i], t2r, sem),
      ]
      for j in range(_R):
        sl = pl.ds(j * _C, _C)
        copies.append(pltpu.async_copy(
            cemb_h.at[c1i.at[sl]], c1r.at[sl], sem))
        copies.append(pltpu.async_copy(
            cemb_h.at[c2i.at[sl]], c2r.at[sl], sem))
        copies.append(pltpu.async_copy(
            pemb_h.at[p1i.at[sl]], p1r.at[sl], sem))
        copies.append(pltpu.async_copy(
            pemb_h.at[p2i.at[sl]], p2r.at[sl], sem))
      for cp in copies:
        cp.wait()

      w0 = [wv[0, pl.ds(16 * v, 16)] for v in range(_NV)]
      w1 = [wv[1, pl.ds(16 * v, 16)] for v in range(_NV)]
      lane = lax.iota(jnp.int32, 16)

      def group(gi, carry2):
        acc0 = jnp.zeros((16,), jnp.float32)
        acc1 = jnp.zeros((16,), jnp.float32)
        for l in range(16):
          e = gi * 16 + l
          feats = [t1r[e, pl.ds(0, 16)], t1r[e, pl.ds(16, 16)],
                   t2r[e, pl.ds(0, 16)], t2r[e, pl.ds(16, 16)]]
          eb = e * _R
          for ref in (c1r, c2r, p1r, p2r):
            lo = ref[eb, pl.ds(0, 16)]
            hi = ref[eb, pl.ds(16, 16)]
            for r in range(1, _R):
              lo = lo + ref[eb + r, pl.ds(0, 16)]
              hi = hi + ref[eb + r, pl.ds(16, 16)]
            feats.append(lo)
            feats.append(hi)
          s0 = feats[0] * w0[0]
          s1 = feats[0] * w1[0]
          for v in range(1, _NV):
            s0 = s0 + feats[v] * w0[v]
            s1 = s1 + feats[v] * w1[v]
          acc0 = jnp.where(lane == l, _lanesum(s0), acc0)
          acc1 = jnp.where(lane == l, _lanesum(s1), acc1)
        ob[0, pl.ds(gi * 16, 16)] = acc0
        ob[1, pl.ds(gi * 16, 16)] = acc1
        return carry2

      lax.fori_loop(0, _C // 16, group, 0)
      pltpu.sync_copy(ob, out_h.at[:, pl.ds(base, _C)])
      return carry

    lax.fori_loop(0, _NCHUNK, chunk_body, 0)

  return k


_tr_kernel = _build_transpose_kernel()
_sc_kernel = _build_gather_kernel()


def kernel(team1_id, team2_id, champions_team1, champions_team2,
           players_team1, players_team2, team_emb, champ_emb, player_emb,
           fc_w, fc_b):
  pt = player_emb.T
  tt = team_emb.T
  ptail = player_emb[_PNB * _TBLK:].reshape(-1)
  ttail = team_emb[_TNB * _TBLK:].reshape(-1)
  p_flat, t_flat = _tr_kernel(pt, tt, ptail, ttail)
  ptab = p_flat.reshape(_PN, _D)
  ttab = t_flat.reshape(_TN, _D)

  c1 = champions_team1.reshape(-1)
  c2 = champions_team2.reshape(-1)
  p1 = players_team1.reshape(-1)
  p2 = players_team2.reshape(-1)

  scale = jnp.concatenate([
      jnp.ones((2 * _D,), jnp.float32),
      jnp.full((4 * _D,), 0.2, jnp.float32),
  ])
  w = fc_w * scale[None, :]

  out = _sc_kernel(team1_id, team2_id, c1, c2, p1, p2,
                   ttab, champ_emb, ptab, w)
  return out.T + fc_b[None, :]


# speedup vs baseline: 1.3107x; 1.3107x over previous
import functools

import jax
import jax.numpy as jnp
from jax import lax
from jax.experimental import pallas as pl
from jax.experimental.pallas import tpu as pltpu
from jax.experimental.pallas import tpu_sc as plsc

_BATCH = 16384
_D = 32
_R = 5
_NC = 2
_NS = 16
_NW = _NC * _NS
_EPW = _BATCH // _NW
_C = 128
_NCHUNK = _EPW // _C
_RC = _R * _C
_F = 6 * _D
_NV = _F // 16

_PN = 1000000
_TN = 100000
_TBLK = 512
_PNB = _PN // _TBLK
_PREM = _PN - _PNB * _TBLK
_TNB = _TN // _TBLK
_TREM = _TN - _TNB * _TBLK


def _ceil_div(a, b):
  return -(-a // b)


def _build_transpose_kernel():
  mesh = plsc.VectorSubcoreMesh(core_axis_name="c", subcore_axis_name="s")
  pni2 = _ceil_div(_ceil_div(_PNB, _NW), 2)
  tni2 = _ceil_div(_ceil_div(_TNB, _NW), 2)

  @functools.partial(
      pl.kernel,
      out_type=(jax.ShapeDtypeStruct((_PN * _D,), jnp.float32),
                jax.ShapeDtypeStruct((_TN * _D,), jnp.float32)),
      mesh=mesh,
      compiler_params=pltpu.CompilerParams(
          use_tc_tiling_on_sc=True, needs_layout_passes=False),
      scratch_types=[
          pltpu.VMEM((_D, _TBLK), jnp.float32),
          pltpu.VMEM((_D, _TBLK), jnp.float32),
          pltpu.VMEM((_D * _TBLK,), jnp.float32),
          pltpu.VMEM((_D * _TBLK,), jnp.float32),
          pltpu.VMEM((_TBLK * 33,), jnp.float32),
          pltpu.SemaphoreType.DMA,
          pltpu.SemaphoreType.DMA,
          pltpu.SemaphoreType.DMA,
          pltpu.SemaphoreType.DMA,
      ],
  )
  def k(pt_h, tt_h, ptail_h, ttail_h, pout_h, tout_h,
        inb0, inb1, outb0, outb1, skew, si0, si1, so0, so1):
    wid = lax.axis_index("s") * _NC + lax.axis_index("c")
    inb = (inb0, inb1)
    outb = (outb0, outb1)
    si = (si0, si1)
    so = (so0, so1)
    lane33 = lax.iota(jnp.int32, 16) * 33

    def transpose_block(bi, bo):
      def fbody(f, c):
        base = lane33 + f
        for g in range(_TBLK // 16):
          v = bi[f, pl.ds(16 * g, 16)]
          plsc.store_scatter(skew, [base + g * 16 * 33], v)
        return c
      lax.fori_loop(0, _D, fbody, 0)

      def cbody(ci, c):
        for u in range(16):
          cc = ci * 16 + u
          bo[pl.ds(cc * _D, 16)] = skew[pl.ds(cc * 33, 16)]
          bo[pl.ds(cc * _D + 16, 16)] = skew[pl.ds(cc * 33 + 16, 16)]
        return c
      lax.fori_loop(0, _TBLK // 16, cbody, 0)

    def run_table(src_h, dst_h, nb, ni2):
      def blk(step):
        return jnp.minimum(wid + step * _NW, nb - 1)

      def issue_in(step, b):
        off = pl.multiple_of(blk(step) * _TBLK, _TBLK)
        pltpu.async_copy(src_h.at[:, pl.ds(off, _TBLK)], inb[b], si[b])

      issue_in(0, 0)
      issue_in(1, 1)

      def outer(i, c):
        for b in (0, 1):
          step = 2 * i + b
          pltpu.make_async_copy(
              src_h.at[:, pl.ds(0, _TBLK)], inb[b], si[b]).wait()

          @pl.when(i > 0)
          def _():
            pltpu.make_async_copy(
                outb[b], dst_h.at[pl.ds(0, _TBLK * _D)], so[b]).wait()

          transpose_block(inb[b], outb[b])
          off = pl.multiple_of(blk(step) * _TBLK * _D, _TBLK * _D)
          pltpu.async_copy(outb[b], dst_h.at[pl.ds(off, _TBLK * _D)], so[b])

          @pl.when(i < ni2 - 1)
          def _():
            issue_in(step + 2, b)
        return c

      lax.fori_loop(0, ni2, outer, 0)
      for b in (0, 1):
        pltpu.make_async_copy(
            outb[b], dst_h.at[pl.ds(0, _TBLK * _D)], so[b]).wait()

    run_table(pt_h, pout_h, _PNB, pni2)
    run_table(tt_h, tout_h, _TNB, tni2)

    @pl.when(wid == 5)
    def _():
      n = _PREM * _D
      pltpu.sync_copy(ptail_h, outb0.at[pl.ds(0, n)])
      pltpu.sync_copy(outb0.at[pl.ds(0, n)],
                      pout_h.at[pl.ds(_PNB * _TBLK * _D, n)])

    @pl.when(wid == 6)
    def _():
      n = _TREM * _D
      pltpu.sync_copy(ttail_h, outb1.at[pl.ds(0, n)])
      pltpu.sync_copy(outb1.at[pl.ds(0, n)],
                      tout_h.at[pl.ds(_TNB * _TBLK * _D, n)])

  return k


def _lanesum(v):
  dnums = lax.GatherDimensionNumbers(
      offset_dims=(), collapsed_slice_dims=(0,), start_index_map=(0,))
  lane = lax.iota(jnp.int32, 16)
  for sh in (8, 4, 2, 1):
    perm = lax.bitwise_xor(lane, sh)
    shuf = lax.gather(v, perm[:, None], dnums, slice_sizes=(1,),
                      mode=lax.GatherScatterMode.PROMISE_IN_BOUNDS)
    v = v + shuf
  return v


def _build_gather_kernel():
  mesh = plsc.VectorSubcoreMesh(core_axis_name="c", subcore_axis_name="s")

  @functools.partial(
      pl.kernel,
      out_type=jax.ShapeDtypeStruct((2, _BATCH), jnp.float32),
      mesh=mesh,
      compiler_params=pltpu.CompilerParams(use_tc_tiling_on_sc=False),
      scratch_types=[
          pltpu.VMEM((_C,), jnp.int32),
          pltpu.VMEM((_C,), jnp.int32),
          pltpu.VMEM((_RC,), jnp.int32),
          pltpu.VMEM((_RC,), jnp.int32),
          pltpu.VMEM((_RC,), jnp.int32),
          pltpu.VMEM((_RC,), jnp.int32),
          pltpu.VMEM((_C, _D), jnp.float32),
          pltpu.VMEM((_C, _D), jnp.float32),
          pltpu.VMEM((_RC, _D), jnp.float32),
          pltpu.VMEM((_RC, _D), jnp.float32),
          pltpu.VMEM((_RC, _D), jnp.float32),
          pltpu.VMEM((_RC, _D), jnp.float32),
          pltpu.VMEM((2, _F), jnp.float32),
          pltpu.VMEM((2, _C), jnp.float32),
          pltpu.SemaphoreType.DMA,
      ],
  )
  def k(t1i_h, t2i_h, c1i_h, c2i_h, p1i_h, p2i_h,
        temb_h, cemb_h, pemb_h, w_h, out_h,
        t1i, t2i, c1i, c2i, p1i, p2i,
        t1r, t2r, c1r, c2r, p1r, p2r, wv, ob, sem):
    wid = lax.axis_index("s") * _NC + lax.axis_index("c")
    pltpu.sync_copy(w_h, wv)

    def chunk_body(g, carry):
      base = wid * _EPW + g * _C
      rbase = base * _R

      pltpu.sync_copy(t1i_h.at[pl.ds(base, _C)], t1i)
      pltpu.sync_copy(t2i_h.at[pl.ds(base, _C)], t2i)
      pltpu.sync_copy(c1i_h.at[pl.ds(rbase, _RC)], c1i)
      pltpu.sync_copy(c2i_h.at[pl.ds(rbase, _RC)], c2i)
      pltpu.sync_copy(p1i_h.at[pl.ds(rbase, _RC)], p1i)
      pltpu.sync_copy(p2i_h.at[pl.ds(rbase, _RC)], p2i)

      copies = [
          pltpu.async_copy(temb_h.at[t1i], t1r, sem),
          pltpu.async_copy(temb_h.at[t2i], t2r, sem),
      ]
      for j in range(_R):
        sl = pl.ds(j * _C, _C)
        copies.append(pltpu.async_copy(
            cemb_h.at[c1i.at[sl]], c1r.at[sl], sem))
        copies.append(pltpu.async_copy(
            cemb_h.at[c2i.at[sl]], c2r.at[sl], sem))
        copies.append(pltpu.async_copy(
            pemb_h.at[p1i.at[sl]], p1r.at[sl], sem))
        copies.append(pltpu.async_copy(
            pemb_h.at[p2i.at[sl]], p2r.at[sl], sem))
      for cp in copies:
        cp.wait()

      w0 = [wv[0, pl.ds(16 * v, 16)] for v in range(_NV)]
      w1 = [wv[1, pl.ds(16 * v, 16)] for v in range(_NV)]
      lane = lax.iota(jnp.int32, 16)

      def group(gi, carry2):
        acc0 = jnp.zeros((16,), jnp.float32)
        acc1 = jnp.zeros((16,), jnp.float32)
        for l in range(16):
          e = gi * 16 + l
          feats = [t1r[e, pl.ds(0, 16)], t1r[e, pl.ds(16, 16)],
                   t2r[e, pl.ds(0, 16)], t2r[e, pl.ds(16, 16)]]
          eb = e * _R
          for ref in (c1r, c2r, p1r, p2r):
            lo = ref[eb, pl.ds(0, 16)]
            hi = ref[eb, pl.ds(16, 16)]
            for r in range(1, _R):
              lo = lo + ref[eb + r, pl.ds(0, 16)]
              hi = hi + ref[eb + r, pl.ds(16, 16)]
            feats.append(lo)
            feats.append(hi)
          s0 = feats[0] * w0[0]
          s1 = feats[0] * w1[0]
          for v in range(1, _NV):
            s0 = s0 + feats[v] * w0[v]
            s1 = s1 + feats[v] * w1[v]
          acc0 = jnp.where(lane == l, _lanesum(s0), acc0)
          acc1 = jnp.where(lane == l, _lanesum(s1), acc1)
        ob[0, pl.ds(gi * 16, 16)] = acc0
        ob[1, pl.ds(gi * 16, 16)] = acc1
        return carry2

      lax.fori_loop(0, _C // 16, group, 0)
      pltpu.sync_copy(ob, out_h.at[:, pl.ds(base, _C)])
      return carry

    lax.fori_loop(0, _NCHUNK, chunk_body, 0)

  return k


_tr_kernel = _build_transpose_kernel()
_sc_kernel = _build_gather_kernel()


def kernel(team1_id, team2_id, champions_team1, champions_team2,
           players_team1, players_team2, team_emb, champ_emb, player_emb,
           fc_w, fc_b):
  pt = player_emb.T
  tt = team_emb.T
  ptail = player_emb[_PNB * _TBLK:].reshape(-1)
  ttail = team_emb[_TNB * _TBLK:].reshape(-1)
  p_flat, t_flat = _tr_kernel(pt, tt, ptail, ttail)
  ptab = p_flat.reshape(_PN, _D)
  ttab = t_flat.reshape(_TN, _D)

  c1 = champions_team1.reshape(-1)
  c2 = champions_team2.reshape(-1)
  p1 = players_team1.reshape(-1)
  p2 = players_team2.reshape(-1)

  scale = jnp.concatenate([
      jnp.ones((2 * _D,), jnp.float32),
      jnp.full((4 * _D,), 0.2, jnp.float32),
  ])
  w = fc_w * scale[None, :]

  out = _sc_kernel(team1_id, team2_id, c1, c2, p1, p2,
                   ttab, champ_emb, ptab, w)
  return out.T + fc_b[None, :]


# speedup vs baseline: 1.3128x; 1.0016x over previous
import functools

import jax
import jax.numpy as jnp
from jax import lax
from jax.experimental import pallas as pl
from jax.experimental.pallas import tpu as pltpu
from jax.experimental.pallas import tpu_sc as plsc

_BATCH = 16384
_D = 32
_R = 5
_NC = 2
_NS = 16
_NW = _NC * _NS
_EPW = _BATCH // _NW
_C = 128
_NCHUNK = _EPW // _C
_RC = _R * _C
_F = 6 * _D
_NV = _F // 16

_PN = 1000000
_TN = 100000
_TBLK = 512
_PNB = _PN // _TBLK
_PREM = _PN - _PNB * _TBLK
_TNB = _TN // _TBLK
_TREM = _TN - _TNB * _TBLK


def _ceil_div(a, b):
  return -(-a // b)


def _build_transpose_kernel():
  mesh = plsc.VectorSubcoreMesh(core_axis_name="c", subcore_axis_name="s")
  pni2 = _ceil_div(_ceil_div(_PNB, _NW), 2)
  tni2 = _ceil_div(_ceil_div(_TNB, _NW), 2)

  @functools.partial(
      pl.kernel,
      out_type=(jax.ShapeDtypeStruct((_PN * _D,), jnp.float32),
                jax.ShapeDtypeStruct((_TN * _D,), jnp.float32)),
      mesh=mesh,
      compiler_params=pltpu.CompilerParams(
          use_tc_tiling_on_sc=True, needs_layout_passes=False),
      scratch_types=[
          pltpu.VMEM((_D, _TBLK), jnp.float32),
          pltpu.VMEM((_D, _TBLK), jnp.float32),
          pltpu.VMEM((_D * _TBLK,), jnp.float32),
          pltpu.VMEM((_D * _TBLK,), jnp.float32),
          pltpu.VMEM((_TBLK * 33,), jnp.float32),
          pltpu.SemaphoreType.DMA,
          pltpu.SemaphoreType.DMA,
          pltpu.SemaphoreType.DMA,
          pltpu.SemaphoreType.DMA,
      ],
  )
  def k(pt_h, tt_h, ptail_h, ttail_h, pout_h, tout_h,
        inb0, inb1, outb0, outb1, skew, si0, si1, so0, so1):
    wid = lax.axis_index("s") * _NC + lax.axis_index("c")
    inb = (inb0, inb1)
    outb = (outb0, outb1)
    si = (si0, si1)
    so = (so0, so1)
    lane33 = lax.iota(jnp.int32, 16) * 33

    def transpose_block(bi, bo):
      def fbody(f, c):
        base = lane33 + f
        for g in range(_TBLK // 16):
          v = bi[f, pl.ds(16 * g, 16)]
          plsc.store_scatter(skew, [base + g * 16 * 33], v)
        return c
      lax.fori_loop(0, _D, fbody, 0)

      def cbody(ci, c):
        for u in range(16):
          cc = ci * 16 + u
          bo[pl.ds(cc * _D, 16)] = skew[pl.ds(cc * 33, 16)]
          bo[pl.ds(cc * _D + 16, 16)] = skew[pl.ds(cc * 33 + 16, 16)]
        return c
      lax.fori_loop(0, _TBLK // 16, cbody, 0)

    def run_table(src_h, dst_h, nb, ni2):
      def blk(step):
        return jnp.minimum(wid + step * _NW, nb - 1)

      def issue_in(step, b):
        off = pl.multiple_of(blk(step) * _TBLK, _TBLK)
        for i in range(_D // 8):
          pltpu.async_copy(src_h.at[pl.ds(8 * i, 8), pl.ds(off, _TBLK)],
                           inb[b].at[pl.ds(8 * i, 8), :], si[b])

      issue_in(0, 0)
      issue_in(1, 1)

      def outer(i, c):
        for b in (0, 1):
          step = 2 * i + b
          for ii in range(_D // 8):
            pltpu.make_async_copy(
                src_h.at[pl.ds(8 * ii, 8), pl.ds(0, _TBLK)],
                inb[b].at[pl.ds(8 * ii, 8), :], si[b]).wait()

          @pl.when(i > 0)
          def _():
            pltpu.make_async_copy(
                outb[b], dst_h.at[pl.ds(0, _TBLK * _D)], so[b]).wait()

          transpose_block(inb[b], outb[b])
          off = pl.multiple_of(blk(step) * _TBLK * _D, _TBLK * _D)
          pltpu.async_copy(outb[b], dst_h.at[pl.ds(off, _TBLK * _D)], so[b])

          @pl.when(i < ni2 - 1)
          def _():
            issue_in(step + 2, b)
        return c

      lax.fori_loop(0, ni2, outer, 0)
      for b in (0, 1):
        pltpu.make_async_copy(
            outb[b], dst_h.at[pl.ds(0, _TBLK * _D)], so[b]).wait()

    run_table(pt_h, pout_h, _PNB, pni2)
    run_table(tt_h, tout_h, _TNB, tni2)

    @pl.when(wid == 5)
    def _():
      n = _PREM * _D
      pltpu.sync_copy(ptail_h, outb0.at[pl.ds(0, n)])
      pltpu.sync_copy(outb0.at[pl.ds(0, n)],
                      pout_h.at[pl.ds(_PNB * _TBLK * _D, n)])

    @pl.when(wid == 6)
    def _():
      n = _TREM * _D
      pltpu.sync_copy(ttail_h, outb1.at[pl.ds(0, n)])
      pltpu.sync_copy(outb1.at[pl.ds(0, n)],
                      tout_h.at[pl.ds(_TNB * _TBLK * _D, n)])

  return k


def _lanesum(v):
  dnums = lax.GatherDimensionNumbers(
      offset_dims=(), collapsed_slice_dims=(0,), start_index_map=(0,))
  lane = lax.iota(jnp.int32, 16)
  for sh in (8, 4, 2, 1):
    perm = lax.bitwise_xor(lane, sh)
    shuf = lax.gather(v, perm[:, None], dnums, slice_sizes=(1,),
                      mode=lax.GatherScatterMode.PROMISE_IN_BOUNDS)
    v = v + shuf
  return v


def _build_gather_kernel():
  mesh = plsc.VectorSubcoreMesh(core_axis_name="c", subcore_axis_name="s")

  @functools.partial(
      pl.kernel,
      out_type=jax.ShapeDtypeStruct((2, _BATCH), jnp.float32),
      mesh=mesh,
      compiler_params=pltpu.CompilerParams(use_tc_tiling_on_sc=False),
      scratch_types=[
          pltpu.VMEM((_C,), jnp.int32),
          pltpu.VMEM((_C,), jnp.int32),
          pltpu.VMEM((_RC,), jnp.int32),
          pltpu.VMEM((_RC,), jnp.int32),
          pltpu.VMEM((_RC,), jnp.int32),
          pltpu.VMEM((_RC,), jnp.int32),
          pltpu.VMEM((_C, _D), jnp.float32),
          pltpu.VMEM((_C, _D), jnp.float32),
          pltpu.VMEM((_RC, _D), jnp.float32),
          pltpu.VMEM((_RC, _D), jnp.float32),
          pltpu.VMEM((_RC, _D), jnp.float32),
          pltpu.VMEM((_RC, _D), jnp.float32),
          pltpu.VMEM((2, _F), jnp.float32),
          pltpu.VMEM((2, _C), jnp.float32),
          pltpu.SemaphoreType.DMA,
      ],
  )
  def k(t1i_h, t2i_h, c1i_h, c2i_h, p1i_h, p2i_h,
        temb_h, cemb_h, pemb_h, w_h, out_h,
        t1i, t2i, c1i, c2i, p1i, p2i,
        t1r, t2r, c1r, c2r, p1r, p2r, wv, ob, sem):
    wid = lax.axis_index("s") * _NC + lax.axis_index("c")
    pltpu.sync_copy(w_h, wv)

    def chunk_body(g, carry):
      base = wid * _EPW + g * _C
      rbase = base * _R

      pltpu.sync_copy(t1i_h.at[pl.ds(base, _C)], t1i)
      pltpu.sync_copy(t2i_h.at[pl.ds(base, _C)], t2i)
      pltpu.sync_copy(c1i_h.at[pl.ds(rbase, _RC)], c1i)
      pltpu.sync_copy(c2i_h.at[pl.ds(rbase, _RC)], c2i)
      pltpu.sync_copy(p1i_h.at[pl.ds(rbase, _RC)], p1i)
      pltpu.sync_copy(p2i_h.at[pl.ds(rbase, _RC)], p2i)

      copies = [
          pltpu.async_copy(temb_h.at[t1i], t1r, sem),
          pltpu.async_copy(temb_h.at[t2i], t2r, sem),
      ]
      for j in range(_R):
        sl = pl.ds(j * _C, _C)
        copies.append(pltpu.async_copy(
            cemb_h.at[c1i.at[sl]], c1r.at[sl], sem))
        copies.append(pltpu.async_copy(
            cemb_h.at[c2i.at[sl]], c2r.at[sl], sem))
        copies.append(pltpu.async_copy(
            pemb_h.at[p1i.at[sl]], p1r.at[sl], sem))
        copies.append(pltpu.async_copy(
            pemb_h.at[p2i.at[sl]], p2r.at[sl], sem))
      for cp in copies:
        cp.wait()

      w0 = [wv[0, pl.ds(16 * v, 16)] for v in range(_NV)]
      w1 = [wv[1, pl.ds(16 * v, 16)] for v in range(_NV)]
      lane = lax.iota(jnp.int32, 16)

      def group(gi, carry2):
        acc0 = jnp.zeros((16,), jnp.float32)
        acc1 = jnp.zeros((16,), jnp.float32)
        for l in range(16):
          e = gi * 16 + l
          feats = [t1r[e, pl.ds(0, 16)], t1r[e, pl.ds(16, 16)],
                   t2r[e, pl.ds(0, 16)], t2r[e, pl.ds(16, 16)]]
          eb = e * _R
          for ref in (c1r, c2r, p1r, p2r):
            lo = ref[eb, pl.ds(0, 16)]
            hi = ref[eb, pl.ds(16, 16)]
            for r in range(1, _R):
              lo = lo + ref[eb + r, pl.ds(0, 16)]
              hi = hi + ref[eb + r, pl.ds(16, 16)]
            feats.append(lo)
            feats.append(hi)
          s0 = feats[0] * w0[0]
          s1 = feats[0] * w1[0]
          for v in range(1, _NV):
            s0 = s0 + feats[v] * w0[v]
            s1 = s1 + feats[v] * w1[v]
          acc0 = jnp.where(lane == l, _lanesum(s0), acc0)
          acc1 = jnp.where(lane == l, _lanesum(s1), acc1)
        ob[0, pl.ds(gi * 16, 16)] = acc0
        ob[1, pl.ds(gi * 16, 16)] = acc1
        return carry2

      lax.fori_loop(0, _C // 16, group, 0)
      pltpu.sync_copy(ob, out_h.at[:, pl.ds(base, _C)])
      return carry

    lax.fori_loop(0, _NCHUNK, chunk_body, 0)

  return k


_tr_kernel = _build_transpose_kernel()
_sc_kernel = _build_gather_kernel()


def kernel(team1_id, team2_id, champions_team1, champions_team2,
           players_team1, players_team2, team_emb, champ_emb, player_emb,
           fc_w, fc_b):
  pt = player_emb.T
  tt = team_emb.T
  ptail = player_emb[_PNB * _TBLK:].reshape(-1)
  ttail = team_emb[_TNB * _TBLK:].reshape(-1)
  p_flat, t_flat = _tr_kernel(pt, tt, ptail, ttail)
  ptab = p_flat.reshape(_PN, _D)
  ttab = t_flat.reshape(_TN, _D)

  c1 = champions_team1.reshape(-1)
  c2 = champions_team2.reshape(-1)
  p1 = players_team1.reshape(-1)
  p2 = players_team2.reshape(-1)

  scale = jnp.concatenate([
      jnp.ones((2 * _D,), jnp.float32),
      jnp.full((4 * _D,), 0.2, jnp.float32),
  ])
  w = fc_w * scale[None, :]

  out = _sc_kernel(team1_id, team2_id, c1, c2, p1, p2,
                   ttab, champ_emb, ptab, w)
  return out.T + fc_b[None, :]


# speedup vs baseline: 3.1090x; 2.3682x over previous
import functools

import jax
import jax.numpy as jnp
from jax import lax
from jax.experimental import pallas as pl
from jax.experimental.pallas import tpu as pltpu
from jax.experimental.pallas import tpu_sc as plsc

_BATCH = 16384
_D = 32
_R = 5
_NC = 2
_NS = 16
_NW = _NC * _NS
_EPW = _BATCH // _NW
_C = 128
_NCHUNK = _EPW // _C
_RC = _R * _C
_F = 6 * _D
_NV = _F // 16

_PN = 1000000
_TN = 100000
_TBLK = 512
_PNB = _PN // _TBLK
_PREM = _PN - _PNB * _TBLK
_TNB = _TN // _TBLK
_TREM = _TN - _TNB * _TBLK


def _ceil_div(a, b):
  return -(-a // b)


def _build_transpose_kernel():
  mesh = plsc.VectorSubcoreMesh(core_axis_name="c", subcore_axis_name="s")
  pni2 = _ceil_div(_ceil_div(_PNB, _NW), 2)
  tni2 = _ceil_div(_ceil_div(_TNB, _NW), 2)

  @functools.partial(
      pl.kernel,
      out_type=(jax.ShapeDtypeStruct((_PN * _D,), jnp.float32),
                jax.ShapeDtypeStruct((_TN * _D,), jnp.float32)),
      mesh=mesh,
      compiler_params=pltpu.CompilerParams(
          use_tc_tiling_on_sc=True, needs_layout_passes=False),
      scratch_types=[
          pltpu.VMEM((_D, _TBLK), jnp.float32),
          pltpu.VMEM((_D, _TBLK), jnp.float32),
          pltpu.VMEM((_D * _TBLK,), jnp.float32),
          pltpu.VMEM((_D * _TBLK,), jnp.float32),
          pltpu.VMEM((_TBLK * 33,), jnp.float32),
          pltpu.SemaphoreType.DMA,
          pltpu.SemaphoreType.DMA,
          pltpu.SemaphoreType.DMA,
          pltpu.SemaphoreType.DMA,
      ],
  )
  def k(pt_h, tt_h, ptail_h, ttail_h, pout_h, tout_h,
        inb0, inb1, outb0, outb1, skew, si0, si1, so0, so1):
    wid = lax.axis_index("s") * _NC + lax.axis_index("c")
    inb = (inb0, inb1)
    outb = (outb0, outb1)
    si = (si0, si1)
    so = (so0, so1)
    lane33 = lax.iota(jnp.int32, 16) * 33

    def transpose_block(bi, bo):
      @functools.partial(plsc.parallel_loop, 0, _D, unroll=2)
      def fbody(f):
        base = lane33 + f
        for g in range(_TBLK // 16):
          v = bi[f, pl.ds(16 * g, 16)]
          plsc.store_scatter(skew, [base + g * 16 * 33], v)

      @functools.partial(plsc.parallel_loop, 0, _TBLK // 16, unroll=2)
      def cbody(ci):
        for u in range(16):
          cc = ci * 16 + u
          bo[pl.ds(cc * _D, 16)] = skew[pl.ds(cc * 33, 16)]
          bo[pl.ds(cc * _D + 16, 16)] = skew[pl.ds(cc * 33 + 16, 16)]

    def run_table(src_h, dst_h, nb, ni2):
      def blk(step):
        return jnp.minimum(wid + step * _NW, nb - 1)

      def issue_in(step, b):
        off = pl.multiple_of(blk(step) * _TBLK, _TBLK)
        for i in range(_D // 8):
          pltpu.async_copy(src_h.at[pl.ds(8 * i, 8), pl.ds(off, _TBLK)],
                           inb[b].at[pl.ds(8 * i, 8), :], si[b])

      issue_in(0, 0)
      issue_in(1, 1)

      def outer(i, c):
        for b in (0, 1):
          step = 2 * i + b
          for ii in range(_D // 8):
            pltpu.make_async_copy(
                src_h.at[pl.ds(8 * ii, 8), pl.ds(0, _TBLK)],
                inb[b].at[pl.ds(8 * ii, 8), :], si[b]).wait()

          @pl.when(i > 0)
          def _():
            pltpu.make_async_copy(
                outb[b], dst_h.at[pl.ds(0, _TBLK * _D)], so[b]).wait()

          transpose_block(inb[b], outb[b])
          off = pl.multiple_of(blk(step) * _TBLK * _D, _TBLK * _D)
          pltpu.async_copy(outb[b], dst_h.at[pl.ds(off, _TBLK * _D)], so[b])

          @pl.when(i < ni2 - 1)
          def _():
            issue_in(step + 2, b)
        return c

      lax.fori_loop(0, ni2, outer, 0)
      for b in (0, 1):
        pltpu.make_async_copy(
            outb[b], dst_h.at[pl.ds(0, _TBLK * _D)], so[b]).wait()

    run_table(pt_h, pout_h, _PNB, pni2)
    run_table(tt_h, tout_h, _TNB, tni2)

    @pl.when(wid == 5)
    def _():
      n = _PREM * _D
      pltpu.sync_copy(ptail_h, outb0.at[pl.ds(0, n)])
      pltpu.sync_copy(outb0.at[pl.ds(0, n)],
                      pout_h.at[pl.ds(_PNB * _TBLK * _D, n)])

    @pl.when(wid == 6)
    def _():
      n = _TREM * _D
      pltpu.sync_copy(ttail_h, outb1.at[pl.ds(0, n)])
      pltpu.sync_copy(outb1.at[pl.ds(0, n)],
                      tout_h.at[pl.ds(_TNB * _TBLK * _D, n)])

  return k


def _lanesum(v):
  dnums = lax.GatherDimensionNumbers(
      offset_dims=(), collapsed_slice_dims=(0,), start_index_map=(0,))
  lane = lax.iota(jnp.int32, 16)
  for sh in (8, 4, 2, 1):
    perm = lax.bitwise_xor(lane, sh)
    shuf = lax.gather(v, perm[:, None], dnums, slice_sizes=(1,),
                      mode=lax.GatherScatterMode.PROMISE_IN_BOUNDS)
    v = v + shuf
  return v


def _build_gather_kernel():
  mesh = plsc.VectorSubcoreMesh(core_axis_name="c", subcore_axis_name="s")

  @functools.partial(
      pl.kernel,
      out_type=jax.ShapeDtypeStruct((2, _BATCH), jnp.float32),
      mesh=mesh,
      compiler_params=pltpu.CompilerParams(use_tc_tiling_on_sc=False),
      scratch_types=[
          pltpu.VMEM((_C,), jnp.int32),
          pltpu.VMEM((_C,), jnp.int32),
          pltpu.VMEM((_RC,), jnp.int32),
          pltpu.VMEM((_RC,), jnp.int32),
          pltpu.VMEM((_RC,), jnp.int32),
          pltpu.VMEM((_RC,), jnp.int32),
          pltpu.VMEM((_C, _D), jnp.float32),
          pltpu.VMEM((_C, _D), jnp.float32),
          pltpu.VMEM((_RC, _D), jnp.float32),
          pltpu.VMEM((_RC, _D), jnp.float32),
          pltpu.VMEM((_RC, _D), jnp.float32),
          pltpu.VMEM((_RC, _D), jnp.float32),
          pltpu.VMEM((2, _F), jnp.float32),
          pltpu.VMEM((2, _C), jnp.float32),
          pltpu.SemaphoreType.DMA,
      ],
  )
  def k(t1i_h, t2i_h, c1i_h, c2i_h, p1i_h, p2i_h,
        temb_h, cemb_h, pemb_h, w_h, out_h,
        t1i, t2i, c1i, c2i, p1i, p2i,
        t1r, t2r, c1r, c2r, p1r, p2r, wv, ob, sem):
    wid = lax.axis_index("s") * _NC + lax.axis_index("c")
    pltpu.sync_copy(w_h, wv)

    def chunk_body(g, carry):
      base = wid * _EPW + g * _C
      rbase = base * _R

      pltpu.sync_copy(t1i_h.at[pl.ds(base, _C)], t1i)
      pltpu.sync_copy(t2i_h.at[pl.ds(base, _C)], t2i)
      pltpu.sync_copy(c1i_h.at[pl.ds(rbase, _RC)], c1i)
      pltpu.sync_copy(c2i_h.at[pl.ds(rbase, _RC)], c2i)
      pltpu.sync_copy(p1i_h.at[pl.ds(rbase, _RC)], p1i)
      pltpu.sync_copy(p2i_h.at[pl.ds(rbase, _RC)], p2i)

      copies = [
          pltpu.async_copy(temb_h.at[t1i], t1r, sem),
          pltpu.async_copy(temb_h.at[t2i], t2r, sem),
      ]
      for j in range(_R):
        sl = pl.ds(j * _C, _C)
        copies.append(pltpu.async_copy(
            cemb_h.at[c1i.at[sl]], c1r.at[sl], sem))
        copies.append(pltpu.async_copy(
            cemb_h.at[c2i.at[sl]], c2r.at[sl], sem))
        copies.append(pltpu.async_copy(
            pemb_h.at[p1i.at[sl]], p1r.at[sl], sem))
        copies.append(pltpu.async_copy(
            pemb_h.at[p2i.at[sl]], p2r.at[sl], sem))
      for cp in copies:
        cp.wait()

      w0 = [wv[0, pl.ds(16 * v, 16)] for v in range(_NV)]
      w1 = [wv[1, pl.ds(16 * v, 16)] for v in range(_NV)]
      lane = lax.iota(jnp.int32, 16)

      def group(gi, carry2):
        acc0 = jnp.zeros((16,), jnp.float32)
        acc1 = jnp.zeros((16,), jnp.float32)
        for l in range(16):
          e = gi * 16 + l
          feats = [t1r[e, pl.ds(0, 16)], t1r[e, pl.ds(16, 16)],
                   t2r[e, pl.ds(0, 16)], t2r[e, pl.ds(16, 16)]]
          eb = e * _R
          for ref in (c1r, c2r, p1r, p2r):
            lo = ref[eb, pl.ds(0, 16)]
            hi = ref[eb, pl.ds(16, 16)]
            for r in range(1, _R):
              lo = lo + ref[eb + r, pl.ds(0, 16)]
              hi = hi + ref[eb + r, pl.ds(16, 16)]
            feats.append(lo)
            feats.append(hi)
          s0 = feats[0] * w0[0]
          s1 = feats[0] * w1[0]
          for v in range(1, _NV):
            s0 = s0 + feats[v] * w0[v]
            s1 = s1 + feats[v] * w1[v]
          acc0 = jnp.where(lane == l, _lanesum(s0), acc0)
          acc1 = jnp.where(lane == l, _lanesum(s1), acc1)
        ob[0, pl.ds(gi * 16, 16)] = acc0
        ob[1, pl.ds(gi * 16, 16)] = acc1
        return carry2

      lax.fori_loop(0, _C // 16, group, 0)
      pltpu.sync_copy(ob, out_h.at[:, pl.ds(base, _C)])
      return carry

    lax.fori_loop(0, _NCHUNK, chunk_body, 0)

  return k


_tr_kernel = _build_transpose_kernel()
_sc_kernel = _build_gather_kernel()


def kernel(team1_id, team2_id, champions_team1, champions_team2,
           players_team1, players_team2, team_emb, champ_emb, player_emb,
           fc_w, fc_b):
  pt = player_emb.T
  tt = team_emb.T
  ptail = player_emb[_PNB * _TBLK:].reshape(-1)
  ttail = team_emb[_TNB * _TBLK:].reshape(-1)
  p_flat, t_flat = _tr_kernel(pt, tt, ptail, ttail)
  ptab = p_flat.reshape(_PN, _D)
  ttab = t_flat.reshape(_TN, _D)

  c1 = champions_team1.reshape(-1)
  c2 = champions_team2.reshape(-1)
  p1 = players_team1.reshape(-1)
  p2 = players_team2.reshape(-1)

  scale = jnp.concatenate([
      jnp.ones((2 * _D,), jnp.float32),
      jnp.full((4 * _D,), 0.2, jnp.float32),
  ])
  w = fc_w * scale[None, :]

  out = _sc_kernel(team1_id, team2_id, c1, c2, p1, p2,
                   ttab, champ_emb, ptab, w)
  return out.T + fc_b[None, :]
